# Initial kernel scaffold; baseline (speedup 1.0000x reference)
import jax
import jax.numpy as jnp
from jax.experimental import pallas as pl


def kernel(x, edge_index, batch, W0, b0, g0, be0, W1, b1, g1, be1, W2, b2, g2, be2):
    raise NotImplementedError("write your pallas kernel here")


# speedup vs baseline: 9.3286x; 9.3286x over previous
import functools

import jax
import jax.numpy as jnp
from jax import lax
from jax.experimental import pallas as pl
from jax.experimental.pallas import tpu as pltpu
from jax.experimental.pallas import tpu_sc as plsc

N = 10000
D = 128
B = 64
E = 320000

NPAD = 10112
NCORES = 2
NSUB = 16
NTILES = NCORES * NSUB
K = 128
CHUNKS = 79
EPT = CHUNKS * K
EPAD = NTILES * EPT
ROWS_PT = NPAD // NSUB

@functools.lru_cache(maxsize=None)
def _mesh():
    return plsc.VectorSubcoreMesh(
        core_axis_name="c", subcore_axis_name="s",
        num_cores=NCORES, num_subcores=NSUB)


def _sc_deg_body(dst_hbm, zeros_hbm, ones_hbm, out_hbm, didx_v, ones_v, acc_sh):
    c = lax.axis_index("c")
    s = lax.axis_index("s")
    pltpu.sync_copy(zeros_hbm.at[pl.ds(s * ROWS_PT, ROWS_PT)],
                    acc_sh.at[pl.ds(s * ROWS_PT, ROWS_PT)])
    pltpu.sync_copy(ones_hbm, ones_v)
    plsc.subcore_barrier()

    base = (s * NCORES + c) * EPT

    def body(j, carry):
        pltpu.sync_copy(dst_hbm.at[pl.ds(base + j * K, K)], didx_v)
        pltpu.sync_copy(ones_v, acc_sh.at[didx_v], add=True)
        return carry

    lax.fori_loop(0, CHUNKS, body, 0)
    plsc.subcore_barrier()
    pltpu.sync_copy(acc_sh.at[pl.ds(s * ROWS_PT, ROWS_PT)],
                    out_hbm.at[c, pl.ds(s * ROWS_PT, ROWS_PT)])


@functools.lru_cache(maxsize=None)
def _sc_deg():
    return pl.kernel(
        _sc_deg_body,
        out_type=jax.ShapeDtypeStruct((NCORES, NPAD, 16), jnp.float32),
        mesh=_mesh(),
        compiler_params=pltpu.CompilerParams(use_tc_tiling_on_sc=False),
        scratch_types=[
            pltpu.VMEM((K,), jnp.int32),
            pltpu.VMEM((K, 16), jnp.float32),
            pltpu.VMEM_SHARED((NPAD, 16), jnp.float32),
        ],
    )


def _sc_agg_body(src_hbm, dst_hbm, xs_hbm, zeros_hbm, out_hbm,
                 sidx_v, didx_v, rows_v, acc_sh, sem):
    c = lax.axis_index("c")
    s = lax.axis_index("s")
    pltpu.sync_copy(zeros_hbm.at[pl.ds(s * ROWS_PT, ROWS_PT)],
                    acc_sh.at[pl.ds(s * ROWS_PT, ROWS_PT)])
    plsc.subcore_barrier()

    base = (s * NCORES + c) * EPT

    def body(j, carry):
        off = base + j * K
        pltpu.sync_copy(src_hbm.at[pl.ds(off, K)], sidx_v)
        pltpu.sync_copy(dst_hbm.at[pl.ds(off, K)], didx_v)
        pltpu.async_copy(xs_hbm.at[sidx_v], rows_v, sem).wait()
        pltpu.sync_copy(rows_v, acc_sh.at[didx_v], add=True)
        return carry

    lax.fori_loop(0, CHUNKS, body, 0)
    plsc.subcore_barrier()
    pltpu.sync_copy(acc_sh.at[pl.ds(s * ROWS_PT, ROWS_PT)],
                    out_hbm.at[c, pl.ds(s * ROWS_PT, ROWS_PT)])


@functools.lru_cache(maxsize=None)
def _sc_agg():
    return pl.kernel(
        _sc_agg_body,
        out_type=jax.ShapeDtypeStruct((NCORES, NPAD, D), jnp.float32),
        mesh=_mesh(),
        scratch_types=[
            pltpu.VMEM((K,), jnp.int32),
            pltpu.VMEM((K,), jnp.int32),
            pltpu.VMEM((K, D), jnp.float32),
            pltpu.VMEM_SHARED((NPAD, D), jnp.float32),
            pltpu.SemaphoreType.DMA,
        ],
    )


def _dinv_from(deg_ref):
    dcol = deg_ref[0, :, 0:1] + deg_ref[1, :, 0:1] + 1.0
    return lax.rsqrt(dcol)[:N]


def _tc0_body(x_ref, w_ref, deg_ref, out_ref):
    dinv = _dinv_from(deg_ref)
    xw = jnp.dot(x_ref[...], w_ref[...], preferred_element_type=jnp.float32)
    out_ref[...] = xw * dinv


_tc0 = pl.pallas_call(
    _tc0_body,
    out_shape=jax.ShapeDtypeStruct((N, D), jnp.float32),
)


def _bn_relu(sp_ref, xs_ref, b_ref, g_ref, be_ref, dinv):
    S = sp_ref[0, :N, :] + sp_ref[1, :N, :] + xs_ref[...]
    h = S * dinv + b_ref[...]
    mu = jnp.mean(h, axis=0)
    d = h - mu
    var = jnp.mean(d * d, axis=0)
    hn = d * lax.rsqrt(var + 1e-5) * g_ref[...] + be_ref[...]
    return jnp.maximum(hn, 0.0)


def _tc_mid_body(sp_ref, xs_ref, b_ref, g_ref, be_ref, deg_ref, wn_ref, out_ref):
    dinv = _dinv_from(deg_ref)
    h2 = _bn_relu(sp_ref, xs_ref, b_ref, g_ref, be_ref, dinv)
    xw = jnp.dot(h2, wn_ref[...], preferred_element_type=jnp.float32)
    out_ref[...] = xw * dinv


_tc_mid = pl.pallas_call(
    _tc_mid_body,
    out_shape=jax.ShapeDtypeStruct((N, D), jnp.float32),
)


def _tc_fin_body(sp_ref, xs_ref, b_ref, g_ref, be_ref, deg_ref, batch_ref, out_ref):
    dinv = _dinv_from(deg_ref)
    h2 = _bn_relu(sp_ref, xs_ref, b_ref, g_ref, be_ref, dinv)
    gids = lax.broadcasted_iota(jnp.int32, (B, N), 0)
    onehot = (batch_ref[...][None, :] == gids).astype(jnp.float32)
    sums = jnp.dot(onehot, h2, preferred_element_type=jnp.float32)
    cnts = jnp.sum(onehot, axis=1)
    out_ref[...] = sums / jnp.maximum(cnts, 1.0)[:, None]


_tc_fin = pl.pallas_call(
    _tc_fin_body,
    out_shape=jax.ShapeDtypeStruct((B, D), jnp.float32),
)


@jax.jit
def kernel(x, edge_index, batch, W0, b0, g0, be0, W1, b1, g1, be1,
           W2, b2, g2, be2):
    npad = EPAD - E
    src_p = jnp.concatenate([edge_index[0], jnp.zeros((npad,), jnp.int32)])
    dst_p = jnp.concatenate([edge_index[1], jnp.full((npad,), N, jnp.int32)])
    z128 = jnp.zeros((NPAD, D), jnp.float32)
    z16 = jnp.zeros((NPAD, 16), jnp.float32)
    o16 = jnp.ones((K, 16), jnp.float32)

    degp = _sc_deg()(dst_p, z16, o16)
    xs = _tc0(x, W0, degp)

    sp = _sc_agg()(src_p, dst_p, xs, z128)
    xs = _tc_mid(sp, xs, b0, g0, be0, degp, W1)
    sp = _sc_agg()(src_p, dst_p, xs, z128)
    xs = _tc_mid(sp, xs, b1, g1, be1, degp, W2)
    sp = _sc_agg()(src_p, dst_p, xs, z128)
    return _tc_fin(sp, xs, b2, g2, be2, degp, batch)
